# baseline (speedup 1.0000x reference)
import dataclasses
import functools

import jax
import jax.numpy as jnp
from jax import lax
from jax.experimental import pallas as pl
from jax.experimental.pallas import tpu as pltpu
from jax.experimental.pallas import tpu_sc as plsc

B = 1024
L = 200
D = 64
NC = 2
NS = 16
NW = NC * NS
ROWS_PER_W = B // NW
RB = 2
NBLK = ROWS_PER_W // RB
BLK = RB * L
LANES = 16


V = 1000000
NCOL = V // 128
VTAIL = V - NCOL * 128
VPAD = (NCOL + 1) * 128


def _transpose_table(table_t, tail_pad):
    mesh = plsc.VectorSubcoreMesh(core_axis_name="c", subcore_axis_name="s")

    @functools.partial(
        pl.kernel,
        out_type=jax.ShapeDtypeStruct((VPAD, 128), jnp.float32),
        mesh=mesh,
        compiler_params=dataclasses.replace(
            pltpu.CompilerParams(use_tc_tiling_on_sc=True),
            needs_layout_passes=False),
        scratch_types=[
            pltpu.VMEM((D, 128), jnp.float32),
            pltpu.VMEM((D, 128), jnp.float32),
            pltpu.VMEM((128, 128), jnp.float32),
            pltpu.VMEM((128, 128), jnp.float32),
            pltpu.SemaphoreType.DMA,
            pltpu.SemaphoreType.DMA,
            pltpu.SemaphoreType.DMA,
            pltpu.SemaphoreType.DMA,
        ],
    )
    def tr_kernel(tt_hbm, tail_hbm, out_hbm, wa, wb, ta, tb, ia, ib, oa, ob):
        wid = lax.axis_index("s") * NC + lax.axis_index("c")
        ncols = (NCOL - wid + NW - 1) // NW

        wbufs, tbufs = (wa, wb), (ta, tb)
        isems, osems = (ia, ib), (oa, ob)

        def issue_in(c, w, isem):
            for g in range(D // 8):
                pltpu.async_copy(
                    tt_hbm.at[pl.ds(8 * g, 8), pl.ds(c * 128, 128)],
                    w.at[pl.ds(8 * g, 8)], isem)

        def wait_in(w, isem):
            for g in range(D // 8):
                pltpu.make_async_copy(
                    tt_hbm.at[pl.ds(0, 8), pl.ds(0, 128)],
                    w.at[pl.ds(8 * g, 8)], isem).wait()

        def wait_out(t, osem):
            pltpu.make_async_copy(
                t, out_hbm.at[pl.ds(0, 128)], osem).wait()

        def transpose_block(w, t, width):
            @plsc.parallel_loop(0, D, unroll=4)
            def _row(d):
                col = jnp.full((LANES,), d, jnp.int32)
                for k in range(width // LANES):
                    rows = jax.lax.iota(jnp.int32, LANES) + (k * LANES)
                    x = w.at[d, pl.ds(k * LANES, LANES)][...]
                    plsc.store_scatter(t, [rows, col], x)

        def col_index(i):
            return i * NW + wid

        @pl.when(ncols > 0)
        def _():
            issue_in(col_index(0), wa, ia)

        @pl.loop(0, 2 * ((ncols + 1) // 2), step=2)
        def _cols(i0):
            for s in range(2):
                i = i0 + s
                w, t = wbufs[s], tbufs[s]
                isem, osem = isems[s], osems[s]
                nw_, nt = wbufs[1 - s], tbufs[1 - s]
                nisem, nosem = isems[1 - s], osems[1 - s]

                @pl.when(i < ncols)
                def _():
                    c = col_index(i)

                    @pl.when(i >= 1)
                    def _():
                        wait_out(nt, nosem)

                    @pl.when(i + 1 < ncols)
                    def _():
                        issue_in(col_index(i + 1), nw_, nisem)

                    wait_in(w, isem)
                    transpose_block(w, t, 128)
                    pltpu.async_copy(t, out_hbm.at[pl.ds(c * 128, 128)],
                                     osem)

        @pl.when((ncols > 0) & (ncols % 2 == 1))
        def _():
            wait_out(ta, oa)

        @pl.when((ncols > 0) & (ncols % 2 == 0))
        def _():
            wait_out(tb, ob)

        @pl.when(wid == 0)
        def _():
            for g in range(D // 8):
                pltpu.sync_copy(tail_hbm.at[pl.ds(8 * g, 8)],
                                wa.at[pl.ds(8 * g, 8)])
            transpose_block(wa, ta, 128)
            pltpu.sync_copy(ta, out_hbm.at[pl.ds(NCOL * 128, 128)])

    return tr_kernel(table_t, tail_pad)


def kernel(input, token_table, pos_table):
    idx_flat = input.reshape(B * L)
    tail_pad = jnp.pad(token_table[NCOL * 128:].T, ((0, 0), (0, 128 - VTAIL)))
    tt_pad = _transpose_table(token_table.T, tail_pad)
    mesh = plsc.VectorSubcoreMesh(core_axis_name="c", subcore_axis_name="s")

    @functools.partial(
        pl.kernel,
        out_type=jax.ShapeDtypeStruct((B * L, 2 * D), jnp.float32),
        mesh=mesh,
        compiler_params=pltpu.CompilerParams(use_tc_tiling_on_sc=False),
        scratch_types=[
            pltpu.VMEM((L * ROWS_PER_W,), jnp.int32),
            pltpu.VMEM((L, D), jnp.float32),
            pltpu.VMEM((BLK, 2 * D), jnp.float32),
            pltpu.VMEM((BLK, 2 * D), jnp.float32),
            pltpu.SemaphoreType.DMA,
            pltpu.SemaphoreType.DMA,
            pltpu.SemaphoreType.DMA,
            pltpu.SemaphoreType.DMA,
        ],
    )
    def emb_kernel(idx_hbm, tok_hbm, pos_hbm, out_hbm,
                   idx_v, pos_v, rows_a, rows_b, ga, gb, oa, ob):
        wid = lax.axis_index("s") * NC + lax.axis_index("c")
        base = wid * (L * ROWS_PER_W)
        pltpu.sync_copy(idx_hbm.at[pl.ds(base, L * ROWS_PER_W)], idx_v)
        pltpu.sync_copy(pos_hbm, pos_v)

        bufs = (rows_a, rows_b)
        gsems = (ga, gb)
        osems = (oa, ob)

        def issue_gather(blk, buf, gsem):
            pltpu.async_copy(tok_hbm.at[idx_v.at[pl.ds(blk * BLK, BLK)]],
                             buf, gsem)

        def wait_gather(buf, gsem):
            pltpu.make_async_copy(tok_hbm.at[idx_v.at[pl.ds(0, BLK)]],
                                  buf, gsem).wait()

        def wait_out(buf, osem):
            pltpu.make_async_copy(buf, out_hbm.at[pl.ds(base, BLK)],
                                  osem).wait()

        def add_pos(buf):
            for q in range(RB):
                @plsc.parallel_loop(0, L, unroll=4)
                def _add_row(i):
                    for j in range(0, D, LANES):
                        src = (pl.ds(i, 1), pl.ds(j, LANES))
                        dst = (pl.ds(q * L + i, 1), pl.ds(j, LANES))
                        buf.at[*dst][...] = (
                            buf.at[*dst][...] + pos_v.at[*src][...])

        issue_gather(0, rows_a, ga)

        @pl.loop(0, NBLK, step=2)
        def _blk(b0):
            for t in range(2):
                blk = b0 + t
                buf, gsem, osem = bufs[t], gsems[t], osems[t]
                nbuf, ngsem, nosem = bufs[1 - t], gsems[1 - t], osems[1 - t]

                @pl.when(blk >= 1)
                def _():
                    wait_out(nbuf, nosem)

                @pl.when(blk + 1 < NBLK)
                def _():
                    issue_gather(blk + 1, nbuf, ngsem)

                wait_gather(buf, gsem)
                add_pos(buf)
                pltpu.async_copy(
                    buf, out_hbm.at[pl.ds(base + blk * BLK, BLK)], osem)

        wait_out(rows_b, ob)

    out = emb_kernel(idx_flat, tt_pad, pos_table)
    return out.reshape(B, L, 2 * D)[:, :, :D]


# speedup vs baseline: 1.7472x; 1.7472x over previous
import functools

import jax
import jax.numpy as jnp
from jax import lax
from jax.experimental import pallas as pl
from jax.experimental.pallas import tpu as pltpu
from jax.experimental.pallas import tpu_sc as plsc

B = 1024
L = 200
D = 64
NC = 2
NS = 16
NW = NC * NS
ROWS_PER_W = B // NW
RB = 2
NBLK = ROWS_PER_W // RB
BLK = RB * L
LANES = 16


def kernel(input, token_table, pos_table):
    idx_flat = input.reshape(B * L)
    tt_pad = jnp.pad(token_table, ((0, 0), (0, D)))
    mesh = plsc.VectorSubcoreMesh(core_axis_name="c", subcore_axis_name="s")

    @functools.partial(
        pl.kernel,
        out_type=jax.ShapeDtypeStruct((B * L, 2 * D), jnp.float32),
        mesh=mesh,
        compiler_params=pltpu.CompilerParams(use_tc_tiling_on_sc=False),
        scratch_types=[
            pltpu.VMEM((L * ROWS_PER_W,), jnp.int32),
            pltpu.VMEM((L, D), jnp.float32),
            pltpu.VMEM((BLK, 2 * D), jnp.float32),
            pltpu.VMEM((BLK, 2 * D), jnp.float32),
            pltpu.SemaphoreType.DMA,
            pltpu.SemaphoreType.DMA,
            pltpu.SemaphoreType.DMA,
            pltpu.SemaphoreType.DMA,
        ],
    )
    def emb_kernel(idx_hbm, tok_hbm, pos_hbm, out_hbm,
                   idx_v, pos_v, rows_a, rows_b, ga, gb, oa, ob):
        wid = lax.axis_index("s") * NC + lax.axis_index("c")
        base = wid * (L * ROWS_PER_W)
        pltpu.sync_copy(idx_hbm.at[pl.ds(base, L * ROWS_PER_W)], idx_v)
        pltpu.sync_copy(pos_hbm, pos_v)

        bufs = (rows_a, rows_b)
        gsems = (ga, gb)
        osems = (oa, ob)

        def issue_gather(blk, buf, gsem):
            pltpu.async_copy(tok_hbm.at[idx_v.at[pl.ds(blk * BLK, BLK)]],
                             buf, gsem)

        def wait_gather(buf, gsem):
            pltpu.make_async_copy(tok_hbm.at[idx_v.at[pl.ds(0, BLK)]],
                                  buf, gsem).wait()

        def wait_out(buf, osem):
            pltpu.make_async_copy(buf, out_hbm.at[pl.ds(base, BLK)],
                                  osem).wait()

        def add_pos(buf):
            for q in range(RB):
                @plsc.parallel_loop(0, L, unroll=4)
                def _add_row(i):
                    for j in range(0, D, LANES):
                        src = (pl.ds(i, 1), pl.ds(j, LANES))
                        dst = (pl.ds(q * L + i, 1), pl.ds(j, LANES))
                        buf.at[*dst][...] = (
                            buf.at[*dst][...] + pos_v.at[*src][...])

        issue_gather(0, rows_a, ga)

        @pl.loop(0, NBLK, step=2)
        def _blk(b0):
            for t in range(2):
                blk = b0 + t
                buf, gsem, osem = bufs[t], gsems[t], osems[t]
                nbuf, ngsem, nosem = bufs[1 - t], gsems[1 - t], osems[1 - t]

                @pl.when(blk >= 1)
                def _():
                    wait_out(nbuf, nosem)

                @pl.when(blk + 1 < NBLK)
                def _():
                    issue_gather(blk + 1, nbuf, ngsem)

                wait_gather(buf, gsem)
                add_pos(buf)
                pltpu.async_copy(
                    buf, out_hbm.at[pl.ds(base + blk * BLK, BLK)], osem)

        wait_out(rows_b, ob)

    out = emb_kernel(idx_flat, tt_pad, pos_table)
    return out.reshape(B, L, 2 * D)[:, :, :D]
